# baseline (speedup 1.0000x reference)
import functools

import jax
import jax.numpy as jnp
from jax import lax
from jax.experimental import pallas as pl
from jax.experimental.pallas import tpu as pltpu
from jax.experimental.pallas import tpu_sc as plsc

VOC_DIM = 1000000
EMB_DIM = 64
B_TOT = 4096 * 200

NUM_CORES = 2
NUM_SUBCORES = 16
NW = NUM_CORES * NUM_SUBCORES
BPW = B_TOT // NW
CHUNK = 400
NCHUNK = BPW // CHUNK

_mesh = plsc.VectorSubcoreMesh(core_axis_name="c", subcore_axis_name="s")


PAD_DIM = 128
NBUF = 2


@functools.partial(
    pl.kernel,
    out_type=jax.ShapeDtypeStruct((B_TOT, PAD_DIM), jnp.float32),
    mesh=_mesh,
    scratch_types=[
        pltpu.VMEM((NBUF, CHUNK), jnp.int32),
        pltpu.VMEM((NBUF, CHUNK, PAD_DIM), jnp.float32),
        [pltpu.SemaphoreType.DMA] * NBUF,
        [pltpu.SemaphoreType.DMA] * NBUF,
        [pltpu.SemaphoreType.DMA] * NBUF,
    ],
    compiler_params=pltpu.CompilerParams(use_tc_tiling_on_sc=False),
)
def _sc_gather(idx_hbm, table_hbm, out_hbm, idx_v, rows_v, isems, gsems, osems):
    wid = lax.axis_index("s") * NUM_CORES + lax.axis_index("c")
    base = wid * BPW

    def start_idx(i, b):
        off = base + i * CHUNK
        pltpu.async_copy(idx_hbm.at[pl.ds(off, CHUNK)], idx_v.at[b], isems[b])

    def start_gather(b):
        pltpu.async_copy(table_hbm.at[idx_v.at[b]], rows_v.at[b], gsems[b])

    def start_out(i, b):
        off = base + i * CHUNK
        pltpu.async_copy(rows_v.at[b], out_hbm.at[pl.ds(off, CHUNK)], osems[b])

    def wait_idx(b):
        pltpu.make_async_copy(
            idx_hbm.at[pl.ds(0, CHUNK)], idx_v.at[b], isems[b]
        ).wait()

    def wait_rows(sems, b):
        pltpu.make_async_copy(
            out_hbm.at[pl.ds(0, CHUNK)], rows_v.at[b], sems[b]
        ).wait()

    start_idx(0, 0)
    start_idx(1, 1)
    wait_idx(0)
    start_gather(0)

    def body(g, carry):
        for b in range(NBUF):
            i = g * NBUF + b
            nb = (b + 1) % NBUF
            wait_rows(gsems, b)
            start_out(i, b)

            @pl.when(i + 1 < NCHUNK)
            def _():
                wait_idx(nb)

                @pl.when(i + 1 >= NBUF)
                def _():
                    wait_rows(osems, nb)
                start_gather(nb)

            @pl.when(i + 2 < NCHUNK)
            def _():
                start_idx(i + 2, b)
        return carry

    lax.fori_loop(0, NCHUNK // NBUF, body, 0)
    for b in range(NBUF):
        wait_rows(osems, b)


SEQ = 200
NBLK = 4096 // PAD_DIM
UNITS = SEQ * NBLK
UPW = UNITS // NW
GW = PAD_DIM


@functools.partial(
    pl.kernel,
    out_type=jax.ShapeDtypeStruct((SEQ, 8, NBLK, 8, PAD_DIM), jnp.float32),
    mesh=_mesh,
    scratch_types=[
        pltpu.VMEM((NBUF, GW), jnp.int32),
        pltpu.VMEM((NBUF, GW, PAD_DIM), jnp.float32),
        pltpu.VMEM((NBUF, EMB_DIM, PAD_DIM), jnp.float32),
        [pltpu.SemaphoreType.DMA] * NBUF,
        [pltpu.SemaphoreType.DMA] * NBUF,
        [pltpu.SemaphoreType.DMA] * NBUF,
    ],
    compiler_params=pltpu.CompilerParams(
        use_tc_tiling_on_sc=False, needs_layout_passes=False
    ),
)
def _sc_gather_t(idx_hbm, table_hbm, out_hbm, idx_v, g_v, o_v, isems, gsems, osems):
    wid = lax.axis_index("s") * NUM_CORES + lax.axis_index("c")
    base = wid * UPW
    rowvecs = [lax.iota(jnp.int32, 16) + 16 * lb for lb in range(8)]
    mvecs = [jnp.remainder(lax.iota(jnp.int32, 16) + k, 16) for k in range(16)]

    def start_idx(t, b):
        off = (base + t) * GW
        pltpu.async_copy(idx_hbm.at[pl.ds(off, GW)], idx_v.at[b], isems[b])

    def wait_idx(b):
        pltpu.make_async_copy(
            idx_hbm.at[pl.ds(0, GW)], idx_v.at[b], isems[b]
        ).wait()

    def start_gather(b):
        pltpu.async_copy(table_hbm.at[idx_v.at[b]], g_v.at[b], gsems[b])

    def wait_gather(b):
        pltpu.make_async_copy(
            table_hbm.at[pl.ds(0, GW)], g_v.at[b], gsems[b]
        ).wait()

    def start_out(t, b):
        u = base + t
        s = u // NBLK
        c = lax.rem(u, NBLK)
        for g in range(8):
            pltpu.async_copy(
                o_v.at[b].at[pl.ds(8 * g, 8)], out_hbm.at[s, g, c], osems[b]
            )

    def wait_out(b):
        pltpu.make_async_copy(
            table_hbm.at[pl.ds(0, EMB_DIM)], o_v.at[b], osems[b]
        ).wait()

    start_idx(0, 0)
    start_idx(1, 1)
    wait_idx(0)
    start_gather(0)

    def body(grp, carry):
        for b in range(NBUF):
            t = grp * NBUF + b
            nb = (b + 1) % NBUF
            wait_gather(b)

            @pl.when(t + 1 < UPW)
            def _():
                wait_idx(nb)
                start_gather(nb)

            @pl.when(t + 2 < UPW)
            def _():
                start_idx(t + 2, b)

            @pl.when(t >= NBUF)
            def _():
                wait_out(b)

            gb = g_v.at[b]
            ob = o_v.at[b]
            iota16 = rowvecs[0]

            def blk_body(blk, c):
                c0 = (blk // 8) * 16
                rv = iota16 + (blk % 8) * 16
                cvs = [mvecs[k] + c0 for k in range(16)]
                vals = [plsc.load_gather(gb, [rv, cvs[k]]) for k in range(16)]
                for k in range(16):
                    plsc.store_scatter(ob, [cvs[k], rv], vals[k])
                return c

            lax.fori_loop(0, 32, blk_body, 0)
            start_out(t, b)
        return carry

    lax.fori_loop(0, UPW // NBUF, body, 0)
    for b in range(NBUF):
        wait_out(b)


BV = 1024


def _tc_pad_body(in_ref, out_ref):
    out_ref[:, 0:EMB_DIM] = in_ref[...]


_tc_pad = pl.pallas_call(
    _tc_pad_body,
    grid=((VOC_DIM + BV - 1) // BV,),
    in_specs=[pl.BlockSpec((BV, EMB_DIM), lambda j: (j, 0))],
    out_specs=pl.BlockSpec((BV, PAD_DIM), lambda j: (j, 0)),
    out_shape=jax.ShapeDtypeStruct((VOC_DIM, PAD_DIM), jnp.float32),
)


def kernel(tok_ids, word_emb):
    idx_t = jnp.transpose(tok_ids).reshape(-1).astype(jnp.int32)
    wpad = _tc_pad(word_emb)
    out5 = _sc_gather_t(idx_t, wpad)
    out = out5.transpose(2, 4, 0, 1, 3).reshape(4096, SEQ, EMB_DIM)
    return out


# speedup vs baseline: 1.7564x; 1.7564x over previous
import functools

import jax
import jax.numpy as jnp
from jax import lax
from jax.experimental import pallas as pl
from jax.experimental.pallas import tpu as pltpu
from jax.experimental.pallas import tpu_sc as plsc

VOC_DIM = 1000000
EMB_DIM = 64
B_TOT = 4096 * 200

NUM_CORES = 2
NUM_SUBCORES = 16
NW = NUM_CORES * NUM_SUBCORES
BPW = B_TOT // NW
CHUNK = 400
NCHUNK = BPW // CHUNK

_mesh = plsc.VectorSubcoreMesh(core_axis_name="c", subcore_axis_name="s")


PAD_DIM = 128
NBUF = 2


@functools.partial(
    pl.kernel,
    out_type=jax.ShapeDtypeStruct((B_TOT, PAD_DIM), jnp.float32),
    mesh=_mesh,
    scratch_types=[
        pltpu.VMEM((NBUF, CHUNK), jnp.int32),
        pltpu.VMEM((NBUF, CHUNK, PAD_DIM), jnp.float32),
        [pltpu.SemaphoreType.DMA] * NBUF,
        [pltpu.SemaphoreType.DMA] * NBUF,
        [pltpu.SemaphoreType.DMA] * NBUF,
    ],
    compiler_params=pltpu.CompilerParams(use_tc_tiling_on_sc=False),
)
def _sc_gather(idx_hbm, table_hbm, out_hbm, idx_v, rows_v, isems, gsems, osems):
    wid = lax.axis_index("s") * NUM_CORES + lax.axis_index("c")
    base = wid * BPW

    def start_idx(i, b):
        off = base + i * CHUNK
        pltpu.async_copy(idx_hbm.at[pl.ds(off, CHUNK)], idx_v.at[b], isems[b])

    def start_gather(b):
        pltpu.async_copy(table_hbm.at[idx_v.at[b]], rows_v.at[b], gsems[b])

    def start_out(i, b):
        off = base + i * CHUNK
        pltpu.async_copy(rows_v.at[b], out_hbm.at[pl.ds(off, CHUNK)], osems[b])

    def wait_idx(b):
        pltpu.make_async_copy(
            idx_hbm.at[pl.ds(0, CHUNK)], idx_v.at[b], isems[b]
        ).wait()

    def wait_rows(sems, b):
        pltpu.make_async_copy(
            out_hbm.at[pl.ds(0, CHUNK)], rows_v.at[b], sems[b]
        ).wait()

    start_idx(0, 0)
    start_idx(1, 1)
    wait_idx(0)
    start_gather(0)

    def body(g, carry):
        for b in range(NBUF):
            i = g * NBUF + b
            nb = (b + 1) % NBUF
            wait_rows(gsems, b)
            start_out(i, b)

            @pl.when(i + 1 < NCHUNK)
            def _():
                wait_idx(nb)

                @pl.when(i + 1 >= NBUF)
                def _():
                    wait_rows(osems, nb)
                start_gather(nb)

            @pl.when(i + 2 < NCHUNK)
            def _():
                start_idx(i + 2, b)
        return carry

    lax.fori_loop(0, NCHUNK // NBUF, body, 0)
    for b in range(NBUF):
        wait_rows(osems, b)


SEQ = 200
NBLK = 4096 // PAD_DIM
UNITS = SEQ * NBLK
UPW = UNITS // NW
GW = PAD_DIM


@functools.partial(
    pl.kernel,
    out_type=jax.ShapeDtypeStruct((SEQ, 8, NBLK, 8, PAD_DIM), jnp.float32),
    mesh=_mesh,
    scratch_types=[
        pltpu.VMEM((NBUF, GW), jnp.int32),
        pltpu.VMEM((NBUF, GW, EMB_DIM), jnp.float32),
        pltpu.VMEM((NBUF, EMB_DIM, PAD_DIM), jnp.float32),
        [pltpu.SemaphoreType.DMA] * NBUF,
        [pltpu.SemaphoreType.DMA] * NBUF,
        [pltpu.SemaphoreType.DMA] * NBUF,
    ],
    compiler_params=pltpu.CompilerParams(
        use_tc_tiling_on_sc=False, needs_layout_passes=False
    ),
)
def _sc_gather_t(idx_hbm, table_hbm, out_hbm, idx_v, g_v, o_v, isems, gsems, osems):
    wid = lax.axis_index("s") * NUM_CORES + lax.axis_index("c")
    base = wid * UPW
    rowvecs = [lax.iota(jnp.int32, 16) + 16 * lb for lb in range(8)]

    def start_idx(t, b):
        off = (base + t) * GW
        pltpu.async_copy(idx_hbm.at[pl.ds(off, GW)], idx_v.at[b], isems[b])

    def wait_idx(b):
        pltpu.make_async_copy(
            idx_hbm.at[pl.ds(0, GW)], idx_v.at[b], isems[b]
        ).wait()

    def start_gather(b):
        pltpu.async_copy(table_hbm.at[idx_v.at[b]], g_v.at[b], gsems[b])

    def wait_gather(b):
        pltpu.make_async_copy(
            table_hbm.at[pl.ds(0, GW)], g_v.at[b], gsems[b]
        ).wait()

    def start_out(t, b):
        u = base + t
        s = u // NBLK
        c = lax.rem(u, NBLK)
        for g in range(8):
            pltpu.async_copy(
                o_v.at[b].at[pl.ds(8 * g, 8)], out_hbm.at[s, g, c], osems[b]
            )

    def wait_out(b):
        for g in range(8):
            pltpu.make_async_copy(
                out_hbm.at[0, 0, 0], o_v.at[b, pl.ds(8 * g, 8)], osems[b]
            ).wait()

    start_idx(0, 0)
    start_idx(1, 1)
    wait_idx(0)
    start_gather(0)

    def body(grp, carry):
        for b in range(NBUF):
            t = grp * NBUF + b
            nb = (b + 1) % NBUF
            wait_gather(b)

            @pl.when(t + 1 < UPW)
            def _():
                wait_idx(nb)
                start_gather(nb)

            @pl.when(t + 2 < UPW)
            def _():
                start_idx(t + 2, b)

            @pl.when(t >= NBUF)
            def _():
                wait_out(b)

            gb = g_v.at[b]
            ob = o_v.at[b]
            iota16 = rowvecs[0]

            def col_body(cb, c):
                c0 = cb * 16
                cvs = [
                    jnp.remainder(iota16 + k, 16) + c0 for k in range(16)
                ]

                def blk_body(lb, c2):
                    rv = iota16 + lb * 16
                    vals = [
                        plsc.load_gather(gb, [rv, cvs[k]]) for k in range(16)
                    ]
                    for k in range(16):
                        plsc.store_scatter(ob, [cvs[k], rv], vals[k])
                    return c2

                lax.fori_loop(0, 8, blk_body, 0)
                return c

            lax.fori_loop(0, EMB_DIM // 16, col_body, 0)
            start_out(t, b)
        return carry

    lax.fori_loop(0, UPW // NBUF, body, 0)
    for b in range(NBUF):
        wait_out(b)


def kernel(tok_ids, word_emb):
    idx_t = jnp.transpose(tok_ids).reshape(-1).astype(jnp.int32) * 2
    wpad = jnp.pad(word_emb, ((0, 0), (0, PAD_DIM - EMB_DIM)))
    w2 = wpad.reshape(2 * VOC_DIM, EMB_DIM)
    out5 = _sc_gather_t(idx_t, w2)
    out = out5.transpose(2, 4, 0, 1, 3).reshape(4096, SEQ, EMB_DIM)
    return out
